# baseline (speedup 1.0000x reference)
import jax
import jax.numpy as jnp
import numpy as np
from jax.experimental import pallas as pl
from jax.experimental.pallas import tpu as pltpu

_N_ENT = 1_000_000
_D = 32
_PACK = 128 // _D
_ROWS = _N_ENT // _PACK
_BLK = 2000
_NB = _ROWS // _BLK


def _huber(e):
    ae = jnp.abs(e)
    c = jnp.minimum(ae, 1.0)
    return c * (ae - 0.5 * c)


def _body(ent_ref, old_ent_ref, mask_ref, sel_ref, rel_ref, old_rel_ref,
          out_ref, acc_ref):
    i = pl.program_id(0)

    @pl.when(i == 0)
    def _init():
        hr = _huber(rel_ref[...] - old_rel_ref[...])
        acc_ref[...] = jnp.zeros_like(acc_ref)
        acc_ref[0:1, :] = jnp.sum(hr, axis=0, keepdims=True)

    m4 = mask_ref[0]
    mexp = jnp.dot(m4, sel_ref[...],
                   preferred_element_type=jnp.float32)
    e = (ent_ref[...] - old_ent_ref[...]) * mexp
    h = _huber(e).reshape(_BLK // 8, 8, 128)
    acc_ref[...] += jnp.sum(h, axis=0)

    @pl.when(i == _NB - 1)
    def _fin():
        out_ref[...] = jnp.sum(acc_ref[...], keepdims=True).reshape(1, 1)


def kernel(ent_embeddings, rel_embeddings, old_ent_embeddings,
           old_rel_embeddings, entity_distill_mask):
    ent2 = ent_embeddings.reshape(_ROWS, 128)
    old2 = old_ent_embeddings.reshape(_ROWS, 128)
    rel2 = rel_embeddings.reshape(-1, 128)
    old_rel2 = old_rel_embeddings.reshape(-1, 128)
    mask3 = entity_distill_mask.reshape(_NB, _BLK, _PACK)
    sel = jnp.asarray(np.kron(np.eye(_PACK, dtype=np.float32),
                              np.ones((1, _D), np.float32)))
    out = pl.pallas_call(
        _body,
        grid=(_NB,),
        in_specs=[
            pl.BlockSpec((_BLK, 128), lambda i: (i, 0)),
            pl.BlockSpec((_BLK, 128), lambda i: (i, 0)),
            pl.BlockSpec((1, _BLK, _PACK), lambda i: (i, 0, 0)),
            pl.BlockSpec(sel.shape, lambda i: (0, 0)),
            pl.BlockSpec(rel2.shape, lambda i: (0, 0)),
            pl.BlockSpec(old_rel2.shape, lambda i: (0, 0)),
        ],
        out_specs=pl.BlockSpec((1, 1), lambda i: (0, 0)),
        out_shape=jax.ShapeDtypeStruct((1, 1), jnp.float32),
        scratch_shapes=[pltpu.VMEM((8, 128), jnp.float32)],
    )(ent2, old2, mask3, sel, rel2, old_rel2)
    return out[0, 0]


# speedup vs baseline: 7.8451x; 7.8451x over previous
import jax
import jax.numpy as jnp
from jax.experimental import pallas as pl
from jax.experimental.pallas import tpu as pltpu

_N_ENT = 1_000_000
_D = 32
_BLK = 16384
_NB = -(-_N_ENT // _BLK)


def _huber(e):
    ae = jnp.abs(e)
    c = jnp.minimum(ae, 1.0)
    return c * (ae - 0.5 * c)


def _body(ent_ref, old_ent_ref, mask_ref, rel_ref, old_rel_ref,
          out_ref, acc_ref):
    i = pl.program_id(0)

    @pl.when(i == 0)
    def _init():
        hr = _huber(rel_ref[...] - old_rel_ref[...])
        acc_ref[...] = jnp.zeros_like(acc_ref)
        acc_ref[0:1, 0:1] = jnp.sum(hr, keepdims=True).reshape(1, 1)

    m = mask_ref[0]
    e = (ent_ref[...] - old_ent_ref[...]) * m
    h = jnp.where(m > 0.0, _huber(e), 0.0)
    acc_ref[...] += jnp.sum(h.reshape(_D, _BLK // 128, 128), axis=1)

    @pl.when(i == _NB - 1)
    def _fin():
        out_ref[...] = jnp.sum(acc_ref[...], keepdims=True).reshape(1, 1)


def kernel(ent_embeddings, rel_embeddings, old_ent_embeddings,
           old_rel_embeddings, entity_distill_mask):
    entT = ent_embeddings.T
    oldT = old_ent_embeddings.T
    relT = rel_embeddings.T
    old_relT = old_rel_embeddings.T
    mask_p = jnp.pad(entity_distill_mask, (0, _NB * _BLK - _N_ENT))
    mask3 = mask_p.reshape(_NB, 1, _BLK)
    out = pl.pallas_call(
        _body,
        grid=(_NB,),
        in_specs=[
            pl.BlockSpec((_D, _BLK), lambda i: (0, i)),
            pl.BlockSpec((_D, _BLK), lambda i: (0, i)),
            pl.BlockSpec((1, 1, _BLK), lambda i: (i, 0, 0)),
            pl.BlockSpec(relT.shape, lambda i: (0, 0)),
            pl.BlockSpec(old_relT.shape, lambda i: (0, 0)),
        ],
        out_specs=pl.BlockSpec((1, 1), lambda i: (0, 0)),
        out_shape=jax.ShapeDtypeStruct((1, 1), jnp.float32),
        scratch_shapes=[pltpu.VMEM((_D, 128), jnp.float32)],
    )(entT, oldT, mask3, relT, old_relT)
    return out[0, 0]


# speedup vs baseline: 11.0889x; 1.4135x over previous
import jax
import jax.numpy as jnp
from jax.experimental import pallas as pl
from jax.experimental.pallas import tpu as pltpu

_N_ENT = 1_000_000
_D = 32
_BLK = 16384
_NB = -(-_N_ENT // _BLK)


def _huber(e):
    ae = jnp.abs(e)
    c = jnp.minimum(ae, 1.0)
    return c * (ae - 0.5 * c)


def _body(ent_ref, old_ent_ref, mask_ref, rel_ref, old_rel_ref,
          out_ref, acc_ref):
    i = pl.program_id(0)

    @pl.when(i == 0)
    def _init():
        hr = _huber(rel_ref[...] - old_rel_ref[...])
        acc_ref[...] = jnp.zeros_like(acc_ref)
        acc_ref[0:1, 0:1] = jnp.sum(hr, keepdims=True).reshape(1, 1)

    m8 = jnp.broadcast_to(mask_ref[0], (8, _BLK))
    d = (ent_ref[...] - old_ent_ref[...]).reshape(4, 8, _BLK)
    e = d * m8[None]
    h = jnp.where(m8[None] > 0.0, _huber(e), 0.0)
    acc_ref[...] += jnp.sum(h, axis=0)

    @pl.when(i == _NB - 1)
    def _fin():
        out_ref[...] = jnp.sum(acc_ref[...], keepdims=True).reshape(1, 1)


def kernel(ent_embeddings, rel_embeddings, old_ent_embeddings,
           old_rel_embeddings, entity_distill_mask):
    entT = ent_embeddings.T
    oldT = old_ent_embeddings.T
    relT = rel_embeddings.T
    old_relT = old_rel_embeddings.T
    mask_p = jnp.pad(entity_distill_mask, (0, _NB * _BLK - _N_ENT))
    mask3 = mask_p.reshape(_NB, 1, _BLK)
    out = pl.pallas_call(
        _body,
        grid=(_NB,),
        in_specs=[
            pl.BlockSpec((_D, _BLK), lambda i: (0, i)),
            pl.BlockSpec((_D, _BLK), lambda i: (0, i)),
            pl.BlockSpec((1, 1, _BLK), lambda i: (i, 0, 0)),
            pl.BlockSpec(relT.shape, lambda i: (0, 0)),
            pl.BlockSpec(old_relT.shape, lambda i: (0, 0)),
        ],
        out_specs=pl.BlockSpec((1, 1), lambda i: (0, 0)),
        out_shape=jax.ShapeDtypeStruct((1, 1), jnp.float32),
        scratch_shapes=[pltpu.VMEM((8, _BLK), jnp.float32)],
    )(entT, oldT, mask3, relT, old_relT)
    return out[0, 0]
